# Initial kernel scaffold; baseline (speedup 1.0000x reference)
import jax
import jax.numpy as jnp
from jax.experimental import pallas as pl


def kernel(X_tilde, X, w):
    raise NotImplementedError("write your pallas kernel here")


# speedup vs baseline: 1.9249x; 1.9249x over previous
import functools

import jax
import jax.numpy as jnp
from jax.experimental import pallas as pl

N_DB = 100000
D_FEAT = 128
Q = 1024
K_CONST = 1.0
TOPK = 10

BN = 2048
NB = 49
NP = NB * BN
SLOT = 16
BIG = 3e38


def _wmax_kernel(w_ref, out_ref):
    out_ref[...] = jnp.max(jax.nn.relu(w_ref[...]), keepdims=True)


def _score_topk_kernel(xt_ref, x_ref, w_ref, wmax_ref, vals_ref, idx_ref,
                       wsel_ref):
    j = pl.program_id(0)
    xt = xt_ref[...]
    xb = x_ref[...]
    s = -2.0 * jax.lax.dot_general(
        xt, xb, (((1,), (1,)), ((), ())),
        preferred_element_type=jnp.float32)
    rn = jnp.sum(xb * xb, axis=1)
    wrow = w_ref[0]
    s = s + rn[None, :] + (wmax_ref[0, 0] - wrow) / K_CONST
    lane = jax.lax.broadcasted_iota(jnp.int32, (Q, BN), 1)
    gidx = j * BN + lane
    s = jnp.where(gidx >= N_DB, BIG, s)

    slot = jax.lax.broadcasted_iota(jnp.int32, (Q, SLOT), 1)
    vacc = jnp.full((Q, SLOT), BIG, jnp.float32)
    iacc = jnp.zeros((Q, SLOT), jnp.int32)
    wacc = jnp.zeros((Q, SLOT), jnp.float32)
    for p in range(TOPK):
        m = jnp.min(s, axis=1, keepdims=True)
        am = jnp.min(jnp.where(s == m, lane, NP), axis=1,
                     keepdims=True)
        sel = lane == am
        wv = jnp.sum(jnp.where(sel, jnp.broadcast_to(wrow, (Q, BN)), 0.0),
                     axis=1, keepdims=True)
        vacc = jnp.where(slot == p, m, vacc)
        iacc = jnp.where(slot == p, j * BN + am, iacc)
        wacc = jnp.where(slot == p, wv, wacc)
        s = jnp.where(sel, BIG, s)
    vals_ref[0] = vacc
    idx_ref[0] = iacc
    wsel_ref[0] = wacc


def _merge_kernel(vals_ref, idx_ref, wsel_ref, xt_ref, wmax_ref,
                  dmax_ref, imax_ref):
    v = vals_ref[...]
    i = idx_ref[...]
    wv = wsel_ref[...]
    ncand = NB * SLOT
    lane = jax.lax.broadcasted_iota(jnp.int32, (Q, ncand), 1)
    slot = jax.lax.broadcasted_iota(jnp.int32, (Q, SLOT), 1)
    vacc = jnp.full((Q, SLOT), BIG, jnp.float32)
    iacc = jnp.zeros((Q, SLOT), jnp.int32)
    wacc = jnp.zeros((Q, SLOT), jnp.float32)
    for p in range(TOPK):
        m = jnp.min(v, axis=1, keepdims=True)
        am = jnp.min(jnp.where(v == m, lane, ncand), axis=1,
                     keepdims=True)
        sel = lane == am
        iv = jnp.sum(jnp.where(sel, i, 0), axis=1, keepdims=True)
        wvv = jnp.sum(jnp.where(sel, wv, 0.0), axis=1, keepdims=True)
        vacc = jnp.where(slot == p, m, vacc)
        iacc = jnp.where(slot == p, iv, iacc)
        wacc = jnp.where(slot == p, wvv, wacc)
        v = jnp.where(sel, BIG, v)

    xt = xt_ref[...]
    qn = jnp.sum(xt * xt, axis=1, keepdims=True)
    wmax = wmax_ref[0, 0]
    d2 = vacc + qn - (wmax - wacc) / K_CONST
    f = wacc - K_CONST * jnp.sqrt(jax.nn.relu(d2))
    f = jnp.where(slot >= TOPK, -BIG, f)
    fm = jnp.max(f, axis=1, keepdims=True)
    fam = jnp.min(jnp.where(f == fm, slot, SLOT), axis=1,
                  keepdims=True)
    dmax_ref[...] = fm
    imax_ref[...] = jnp.sum(jnp.where(slot == fam, iacc, 0), axis=1,
                            keepdims=True)


@functools.partial(jax.jit, static_argnames=("interpret",))
def kernel(X_tilde, X, w, interpret=False):
    X_tilde = X_tilde.reshape(Q, D_FEAT)
    Xp = jnp.pad(X, ((0, NP - N_DB), (0, 0)))
    wflat = jnp.pad(w[:, 0], (0, NP - N_DB))
    wrows = wflat.reshape(NB, 1, BN)
    wgrid = wflat.reshape(NP // D_FEAT, D_FEAT)

    wmax = pl.pallas_call(
        _wmax_kernel,
        out_shape=jax.ShapeDtypeStruct((1, 1), jnp.float32),
        interpret=interpret,
    )(wgrid)

    vals, idx, wsel = pl.pallas_call(
        _score_topk_kernel,
        grid=(NB,),
        in_specs=[
            pl.BlockSpec((Q, D_FEAT), lambda j: (0, 0)),
            pl.BlockSpec((BN, D_FEAT), lambda j: (j, 0)),
            pl.BlockSpec((1, 1, BN), lambda j: (j, 0, 0)),
            pl.BlockSpec((1, 1), lambda j: (0, 0)),
        ],
        out_specs=[
            pl.BlockSpec((1, Q, SLOT), lambda j: (j, 0, 0)),
            pl.BlockSpec((1, Q, SLOT), lambda j: (j, 0, 0)),
            pl.BlockSpec((1, Q, SLOT), lambda j: (j, 0, 0)),
        ],
        out_shape=[
            jax.ShapeDtypeStruct((NB, Q, SLOT), jnp.float32),
            jax.ShapeDtypeStruct((NB, Q, SLOT), jnp.int32),
            jax.ShapeDtypeStruct((NB, Q, SLOT), jnp.float32),
        ],
        interpret=interpret,
    )(X_tilde, Xp, wrows, wmax)

    vals = jnp.transpose(vals, (1, 0, 2)).reshape(Q, NB * SLOT)
    idx = jnp.transpose(idx, (1, 0, 2)).reshape(Q, NB * SLOT)
    wsel = jnp.transpose(wsel, (1, 0, 2)).reshape(Q, NB * SLOT)

    dmax, imax = pl.pallas_call(
        _merge_kernel,
        out_shape=[
            jax.ShapeDtypeStruct((Q, 1), jnp.float32),
            jax.ShapeDtypeStruct((Q, 1), jnp.int32),
        ],
        interpret=interpret,
    )(vals, idx, wsel, X_tilde, wmax)

    return dmax[:, 0], imax[:, 0]


# speedup vs baseline: 5.4721x; 2.8429x over previous
import functools

import jax
import jax.numpy as jnp
from jax import lax
from jax.experimental import pallas as pl
from jax.experimental.pallas import tpu as pltpu
from jax.experimental.pallas import tpu_sc as plsc

N_DB = 100000
D_FEAT = 128
Q = 1024
K_CONST = 1.0
TOPK = 10

BN = 2048
NB = 49
NP = NB * BN
NGB = BN // 16
NG = NB * NGB
CAND = TOPK * 16
SLOT = 16
BIG = 3e38

NUM_SC_CORES = 2
NUM_SUBCORES = 16
NW = NUM_SC_CORES * NUM_SUBCORES
EPW = Q * CAND // NW
CHUNK = 128
NCHUNK = EPW // CHUNK


def _wmax_kernel(w_ref, out_ref):
    out_ref[...] = jnp.max(jax.nn.relu(w_ref[...]), keepdims=True)


def _score_kernel(xt_ref, x_ref, w_ref, wmax_ref, s_ref, gm_ref):
    j = pl.program_id(0)
    xt = xt_ref[...]
    xb = x_ref[...]
    s = -2.0 * jax.lax.dot_general(
        xt, xb, (((1,), (1,)), ((), ())),
        preferred_element_type=jnp.float32)
    rn = jnp.sum(xb * xb, axis=1)
    wrow = w_ref[0]
    s = s + rn[None, :] + (wmax_ref[0, 0] - wrow) / K_CONST
    lane = jax.lax.broadcasted_iota(jnp.int32, (Q, BN), 1)
    s = jnp.where(j * BN + lane >= N_DB, BIG, s)
    s_ref[...] = s
    gm = s[:, 0:NGB]
    for c in range(1, 16):
        gm = jnp.minimum(gm, s[:, c * NGB:(c + 1) * NGB])
    gm_ref[...] = gm


QB = 256


def _group_select_kernel(gm_ref, sidx_ref, cidx_ref):
    i = pl.program_id(0)
    gm = gm_ref[...]
    lane = jax.lax.broadcasted_iota(jnp.int32, (QB, NG), 1)
    cl = jax.lax.broadcasted_iota(jnp.int32, (QB, CAND), 1)
    qrow = i * QB + jax.lax.broadcasted_iota(jnp.int32, (QB, CAND), 0)
    e = cl % 16
    cidx = jnp.zeros((QB, CAND), jnp.int32)
    for p in range(TOPK):
        m = jnp.min(gm, axis=1, keepdims=True)
        am = jnp.min(jnp.where(gm == m, lane, NG), axis=1, keepdims=True)
        gm = jnp.where(lane == am, BIG, gm)
        col_p = (am // NGB) * BN + (am % NGB)
        sel = (cl >= p * 16) & (cl < (p + 1) * 16)
        cidx = jnp.where(sel, col_p + NGB * e, cidx)
    cidx_ref[...] = cidx
    sidx_ref[...] = qrow * NP + cidx


def _sc_gather(sidx_hbm, cidx_hbm, s_hbm, w_hbm, sv_hbm, wv_hbm,
               sidx_v, cidx_v, sv_v, wv_v, sem):
    wid = lax.axis_index("s") * NUM_SC_CORES + lax.axis_index("c")
    base = wid * EPW
    pltpu.sync_copy(sidx_hbm.at[pl.ds(base, EPW)], sidx_v)
    pltpu.sync_copy(cidx_hbm.at[pl.ds(base, EPW)], cidx_v)

    def body(i, carry):
        off = i * (4 * CHUNK)
        handles = []
        for t in range(4):
            o = off + t * CHUNK
            handles.append(
                pltpu.async_copy(s_hbm.at[sidx_v.at[pl.ds(o, CHUNK)]],
                                 sv_v.at[pl.ds(o, CHUNK)], sem))
            handles.append(
                pltpu.async_copy(w_hbm.at[cidx_v.at[pl.ds(o, CHUNK)]],
                                 wv_v.at[pl.ds(o, CHUNK)], sem))
        for h in handles:
            h.wait()
        return carry

    lax.fori_loop(0, NCHUNK // 4, body, 0)
    pltpu.sync_copy(sv_v, sv_hbm.at[pl.ds(base, EPW)])
    pltpu.sync_copy(wv_v, wv_hbm.at[pl.ds(base, EPW)])


def _final_kernel(sv_ref, wv_ref, cidx_ref, xt_ref, wmax_ref,
                  dmax_ref, imax_ref):
    v = sv_ref[...]
    wv = wv_ref[...]
    ci = cidx_ref[...]
    lane = jax.lax.broadcasted_iota(jnp.int32, (Q, CAND), 1)
    slot = jax.lax.broadcasted_iota(jnp.int32, (Q, SLOT), 1)
    vacc = jnp.full((Q, SLOT), BIG, jnp.float32)
    iacc = jnp.zeros((Q, SLOT), jnp.int32)
    wacc = jnp.zeros((Q, SLOT), jnp.float32)
    for p in range(TOPK):
        m = jnp.min(v, axis=1, keepdims=True)
        am = jnp.min(jnp.where(v == m, lane, CAND), axis=1, keepdims=True)
        sel = lane == am
        iv = jnp.sum(jnp.where(sel, ci, 0), axis=1, keepdims=True)
        wvv = jnp.sum(jnp.where(sel, wv, 0.0), axis=1, keepdims=True)
        vacc = jnp.where(slot == p, m, vacc)
        iacc = jnp.where(slot == p, iv, iacc)
        wacc = jnp.where(slot == p, wvv, wacc)
        v = jnp.where(sel, BIG, v)

    xt = xt_ref[...]
    qn = jnp.sum(xt * xt, axis=1, keepdims=True)
    wmax = wmax_ref[0, 0]
    d2 = vacc + qn - (wmax - wacc) / K_CONST
    f = wacc - K_CONST * jnp.sqrt(jax.nn.relu(d2))
    f = jnp.where(slot >= TOPK, -BIG, f)
    fm = jnp.max(f, axis=1, keepdims=True)
    fam = jnp.min(jnp.where(f == fm, slot, SLOT), axis=1, keepdims=True)
    dmax_ref[...] = fm
    imax_ref[...] = jnp.sum(jnp.where(slot == fam, iacc, 0), axis=1,
                            keepdims=True)


@jax.jit
def kernel(X_tilde, X, w):
    X_tilde = X_tilde.reshape(Q, D_FEAT)
    Xp = jnp.pad(X, ((0, NP - N_DB), (0, 0)))
    wflat = jnp.pad(w[:, 0], (0, NP - N_DB))
    wrows = wflat.reshape(NB, 1, BN)
    wgrid = wflat.reshape(NP // D_FEAT, D_FEAT)

    wmax = pl.pallas_call(
        _wmax_kernel,
        out_shape=jax.ShapeDtypeStruct((1, 1), jnp.float32),
    )(wgrid)

    s_full, gm = pl.pallas_call(
        _score_kernel,
        grid=(NB,),
        in_specs=[
            pl.BlockSpec((Q, D_FEAT), lambda j: (0, 0)),
            pl.BlockSpec((BN, D_FEAT), lambda j: (j, 0)),
            pl.BlockSpec((1, 1, BN), lambda j: (j, 0, 0)),
            pl.BlockSpec((1, 1), lambda j: (0, 0)),
        ],
        out_specs=[
            pl.BlockSpec((Q, BN), lambda j: (0, j)),
            pl.BlockSpec((Q, NGB), lambda j: (0, j)),
        ],
        out_shape=[
            jax.ShapeDtypeStruct((Q, NP), jnp.float32),
            jax.ShapeDtypeStruct((Q, NG), jnp.float32),
        ],
    )(X_tilde, Xp, wrows, wmax)

    sidx, cidx = pl.pallas_call(
        _group_select_kernel,
        grid=(Q // QB,),
        in_specs=[pl.BlockSpec((QB, NG), lambda i: (i, 0))],
        out_specs=[
            pl.BlockSpec((QB, CAND), lambda i: (i, 0)),
            pl.BlockSpec((QB, CAND), lambda i: (i, 0)),
        ],
        out_shape=[
            jax.ShapeDtypeStruct((Q, CAND), jnp.int32),
            jax.ShapeDtypeStruct((Q, CAND), jnp.int32),
        ],
    )(gm)

    gather = pl.kernel(
        _sc_gather,
        mesh=plsc.VectorSubcoreMesh(core_axis_name="c", subcore_axis_name="s"),
        out_type=[
            jax.ShapeDtypeStruct((Q * CAND,), jnp.float32),
            jax.ShapeDtypeStruct((Q * CAND,), jnp.float32),
        ],
        scratch_types=[
            pltpu.VMEM((EPW,), jnp.int32),
            pltpu.VMEM((EPW,), jnp.int32),
            pltpu.VMEM((EPW,), jnp.float32),
            pltpu.VMEM((EPW,), jnp.float32),
            pltpu.SemaphoreType.DMA,
        ],
    )
    sv, wv = gather(sidx.reshape(Q * CAND), cidx.reshape(Q * CAND),
                    s_full.reshape(Q * NP), wflat)

    dmax, imax = pl.pallas_call(
        _final_kernel,
        out_shape=[
            jax.ShapeDtypeStruct((Q, 1), jnp.float32),
            jax.ShapeDtypeStruct((Q, 1), jnp.int32),
        ],
    )(sv.reshape(Q, CAND), wv.reshape(Q, CAND), cidx, X_tilde, wmax)

    return dmax[:, 0], imax[:, 0]


# speedup vs baseline: 5.7950x; 1.0590x over previous
import functools

import jax
import jax.numpy as jnp
from jax import lax
from jax.experimental import pallas as pl
from jax.experimental.pallas import tpu as pltpu
from jax.experimental.pallas import tpu_sc as plsc

N_DB = 100000
D_FEAT = 128
Q = 1024
K_CONST = 1.0
TOPK = 10

BN = 2048
NB = 49
NP = NB * BN
NGB = BN // 16
NG = NB * NGB
CAND = TOPK * 16
SLOT = 16
BIG = 3e38

NUM_SC_CORES = 2
NUM_SUBCORES = 16
NW = NUM_SC_CORES * NUM_SUBCORES
EPW = Q * CAND // NW
CHUNK = 128
NCHUNK = EPW // CHUNK


def _wmax_kernel(w_ref, out_ref):
    out_ref[...] = jnp.max(jax.nn.relu(w_ref[...]), keepdims=True)


def _score_kernel(xt_ref, x_ref, w_ref, wmax_ref, s_ref, gm_ref):
    j = pl.program_id(0)
    xt = xt_ref[...]
    xb = x_ref[...]
    s = -2.0 * jax.lax.dot_general(
        xt, xb, (((1,), (1,)), ((), ())),
        preferred_element_type=jnp.float32)
    rn = jnp.sum(xb * xb, axis=1)
    wrow = w_ref[0]
    s = s + rn[None, :] + (wmax_ref[0, 0] - wrow) / K_CONST
    lane = jax.lax.broadcasted_iota(jnp.int32, (Q, BN), 1)
    s = jnp.where(j * BN + lane >= N_DB, BIG, s)
    s_ref[...] = s
    gm = s[:, 0:NGB]
    for c in range(1, 16):
        gm = jnp.minimum(gm, s[:, c * NGB:(c + 1) * NGB])
    gm_ref[...] = gm


QB = 256


def _group_select_kernel(gm_ref, sidx_ref, cidx_ref):
    i = pl.program_id(0)
    gm = gm_ref[...]
    lane = jax.lax.broadcasted_iota(jnp.int32, (QB, NG), 1)
    cl = jax.lax.broadcasted_iota(jnp.int32, (QB, CAND), 1)
    qrow = i * QB + jax.lax.broadcasted_iota(jnp.int32, (QB, CAND), 0)
    e = cl % 16
    cidx = jnp.zeros((QB, CAND), jnp.int32)
    for p in range(TOPK):
        m = jnp.min(gm, axis=1, keepdims=True)
        am = jnp.min(jnp.where(gm == m, lane, NG), axis=1, keepdims=True)
        gm = jnp.where(lane == am, BIG, gm)
        col_p = (am // NGB) * BN + (am % NGB)
        sel = (cl >= p * 16) & (cl < (p + 1) * 16)
        cidx = jnp.where(sel, col_p + NGB * e, cidx)
    cidx_ref[...] = cidx
    sidx_ref[...] = qrow * NP + cidx


def _sc_gather(sidx_hbm, cidx_hbm, s_hbm, w_hbm, sv_hbm, wv_hbm,
               sidx_v, cidx_v, sv_v, wv_v, sem):
    wid = lax.axis_index("s") * NUM_SC_CORES + lax.axis_index("c")
    base = wid * EPW
    pltpu.sync_copy(sidx_hbm.at[pl.ds(base, EPW)], sidx_v)
    pltpu.sync_copy(cidx_hbm.at[pl.ds(base, EPW)], cidx_v)

    def body(i, carry):
        off = i * (4 * CHUNK)
        handles = []
        for t in range(4):
            o = off + t * CHUNK
            handles.append(
                pltpu.async_copy(s_hbm.at[sidx_v.at[pl.ds(o, CHUNK)]],
                                 sv_v.at[pl.ds(o, CHUNK)], sem))
            handles.append(
                pltpu.async_copy(w_hbm.at[cidx_v.at[pl.ds(o, CHUNK)]],
                                 wv_v.at[pl.ds(o, CHUNK)], sem))
        for h in handles:
            h.wait()
        return carry

    lax.fori_loop(0, NCHUNK // 4, body, 0)
    pltpu.sync_copy(sv_v, sv_hbm.at[pl.ds(base, EPW)])
    pltpu.sync_copy(wv_v, wv_hbm.at[pl.ds(base, EPW)])


def _final_kernel(sv_ref, wv_ref, cidx_ref, xt_ref, wmax_ref,
                  dmax_ref, imax_ref):
    v = sv_ref[...]
    wv = wv_ref[...]
    ci = cidx_ref[...]
    lane = jax.lax.broadcasted_iota(jnp.int32, (Q, CAND), 1)
    slot = jax.lax.broadcasted_iota(jnp.int32, (Q, SLOT), 1)
    vacc = jnp.full((Q, SLOT), BIG, jnp.float32)
    iacc = jnp.zeros((Q, SLOT), jnp.int32)
    wacc = jnp.zeros((Q, SLOT), jnp.float32)
    for p in range(TOPK):
        m = jnp.min(v, axis=1, keepdims=True)
        am = jnp.min(jnp.where(v == m, lane, CAND), axis=1, keepdims=True)
        sel = lane == am
        iv = jnp.sum(jnp.where(sel, ci, 0), axis=1, keepdims=True)
        wvv = jnp.sum(jnp.where(sel, wv, 0.0), axis=1, keepdims=True)
        vacc = jnp.where(slot == p, m, vacc)
        iacc = jnp.where(slot == p, iv, iacc)
        wacc = jnp.where(slot == p, wvv, wacc)
        v = jnp.where(sel, BIG, v)

    xt = xt_ref[...]
    qn = jnp.sum(xt * xt, axis=1, keepdims=True)
    wmax = wmax_ref[0, 0]
    d2 = vacc + qn - (wmax - wacc) / K_CONST
    f = wacc - K_CONST * jnp.sqrt(jax.nn.relu(d2))
    f = jnp.where(slot >= TOPK, -BIG, f)
    fm = jnp.max(f, axis=1, keepdims=True)
    fam = jnp.min(jnp.where(f == fm, slot, SLOT), axis=1, keepdims=True)
    dmax_ref[...] = fm
    imax_ref[...] = jnp.sum(jnp.where(slot == fam, iacc, 0), axis=1,
                            keepdims=True)


@jax.jit
def kernel(X_tilde, X, w):
    X_tilde = X_tilde.reshape(Q, D_FEAT)
    wflat = jnp.pad(w[:, 0], (0, NP - N_DB))
    wrows = wflat.reshape(NB, 1, BN)
    wgrid = wflat.reshape(NP // D_FEAT, D_FEAT)

    wmax = pl.pallas_call(
        _wmax_kernel,
        out_shape=jax.ShapeDtypeStruct((1, 1), jnp.float32),
    )(wgrid)

    s_full, gm = pl.pallas_call(
        _score_kernel,
        grid=(NB,),
        in_specs=[
            pl.BlockSpec((Q, D_FEAT), lambda j: (0, 0)),
            pl.BlockSpec((BN, D_FEAT), lambda j: (j, 0)),
            pl.BlockSpec((1, 1, BN), lambda j: (j, 0, 0)),
            pl.BlockSpec((1, 1), lambda j: (0, 0)),
        ],
        out_specs=[
            pl.BlockSpec((Q, BN), lambda j: (0, j)),
            pl.BlockSpec((Q, NGB), lambda j: (0, j)),
        ],
        out_shape=[
            jax.ShapeDtypeStruct((Q, NP), jnp.float32),
            jax.ShapeDtypeStruct((Q, NG), jnp.float32),
        ],
    )(X_tilde, X, wrows, wmax)

    sidx, cidx = pl.pallas_call(
        _group_select_kernel,
        grid=(Q // QB,),
        in_specs=[pl.BlockSpec((QB, NG), lambda i: (i, 0))],
        out_specs=[
            pl.BlockSpec((QB, CAND), lambda i: (i, 0)),
            pl.BlockSpec((QB, CAND), lambda i: (i, 0)),
        ],
        out_shape=[
            jax.ShapeDtypeStruct((Q, CAND), jnp.int32),
            jax.ShapeDtypeStruct((Q, CAND), jnp.int32),
        ],
    )(gm)

    gather = pl.kernel(
        _sc_gather,
        mesh=plsc.VectorSubcoreMesh(core_axis_name="c", subcore_axis_name="s"),
        out_type=[
            jax.ShapeDtypeStruct((Q * CAND,), jnp.float32),
            jax.ShapeDtypeStruct((Q * CAND,), jnp.float32),
        ],
        scratch_types=[
            pltpu.VMEM((EPW,), jnp.int32),
            pltpu.VMEM((EPW,), jnp.int32),
            pltpu.VMEM((EPW,), jnp.float32),
            pltpu.VMEM((EPW,), jnp.float32),
            pltpu.SemaphoreType.DMA,
        ],
    )
    sv, wv = gather(sidx.reshape(Q * CAND), cidx.reshape(Q * CAND),
                    s_full.reshape(Q * NP), wflat)

    dmax, imax = pl.pallas_call(
        _final_kernel,
        out_shape=[
            jax.ShapeDtypeStruct((Q, 1), jnp.float32),
            jax.ShapeDtypeStruct((Q, 1), jnp.int32),
        ],
    )(sv.reshape(Q, CAND), wv.reshape(Q, CAND), cidx, X_tilde, wmax)

    return dmax[:, 0], imax[:, 0]


# speedup vs baseline: 6.3178x; 1.0902x over previous
import functools

import jax
import jax.numpy as jnp
from jax import lax
from jax.experimental import pallas as pl
from jax.experimental.pallas import tpu as pltpu
from jax.experimental.pallas import tpu_sc as plsc

N_DB = 100000
D_FEAT = 128
Q = 1024
K_CONST = 1.0
TOPK = 10

BN = 2048
NB = 49
NP = NB * BN
NGB = BN // 16
NG = NB * NGB
CAND = TOPK * 16
SLOT = 16
BIG = 3e38

NUM_SC_CORES = 2
NUM_SUBCORES = 16
NW = NUM_SC_CORES * NUM_SUBCORES
EPW = Q * CAND // NW
CHUNK = 128
NCHUNK = EPW // CHUNK


def _wmax_kernel(w_ref, out_ref):
    out_ref[...] = jnp.max(jax.nn.relu(w_ref[...]), keepdims=True)


def _score_kernel(xt_ref, x_ref, w_ref, wmax_ref, s_ref, gm_ref):
    j = pl.program_id(0)
    xt = xt_ref[...]
    xb = x_ref[...]
    s = -2.0 * jax.lax.dot_general(
        xt, xb, (((1,), (1,)), ((), ())),
        preferred_element_type=jnp.float32)
    rn = jnp.sum(xb * xb, axis=1)
    wrow = w_ref[0]
    s = s + rn[None, :] + (wmax_ref[0, 0] - wrow) / K_CONST
    lane = jax.lax.broadcasted_iota(jnp.int32, (Q, BN), 1)
    s = jnp.where(j * BN + lane >= N_DB, BIG, s)
    for t in range(BN // 128):
        s_ref[:, t, :] = s[:, t * 128:(t + 1) * 128]
    gm = s[:, 0:NGB]
    for c in range(1, 16):
        gm = jnp.minimum(gm, s[:, c * NGB:(c + 1) * NGB])
    gm_ref[...] = gm


QB = 256


def _group_select_kernel(gm_ref, sidx_ref, cidx_ref):
    i = pl.program_id(0)
    gm = gm_ref[...]
    lane = jax.lax.broadcasted_iota(jnp.int32, (QB, NG), 1)
    cl = jax.lax.broadcasted_iota(jnp.int32, (QB, CAND), 1)
    qrow = i * QB + jax.lax.broadcasted_iota(jnp.int32, (QB, CAND), 0)
    e = cl % 16
    cidx = jnp.zeros((QB, CAND), jnp.int32)
    for p in range(TOPK):
        m = jnp.min(gm, axis=1, keepdims=True)
        am = jnp.min(jnp.where(gm == m, lane, NG), axis=1, keepdims=True)
        gm = jnp.where(lane == am, BIG, gm)
        col_p = (am // NGB) * BN + (am % NGB)
        sel = (cl >= p * 16) & (cl < (p + 1) * 16)
        cidx = jnp.where(sel, col_p + NGB * e, cidx)
    cidx_ref[...] = cidx
    sidx_ref[...] = qrow * NP + cidx


def _sc_gather(sidx_hbm, cidx_hbm, s_hbm, w_hbm, sv_hbm, wv_hbm,
               sidx_v, cidx_v, sv_v, wv_v, sem):
    wid = lax.axis_index("s") * NUM_SC_CORES + lax.axis_index("c")
    base = wid * EPW
    pltpu.sync_copy(sidx_hbm.at[pl.ds(base, EPW)], sidx_v)
    pltpu.sync_copy(cidx_hbm.at[pl.ds(base, EPW)], cidx_v)

    def body(i, carry):
        off = i * (4 * CHUNK)
        handles = []
        for t in range(4):
            o = off + t * CHUNK
            handles.append(
                pltpu.async_copy(s_hbm.at[sidx_v.at[pl.ds(o, CHUNK)]],
                                 sv_v.at[pl.ds(o, CHUNK)], sem))
            handles.append(
                pltpu.async_copy(w_hbm.at[cidx_v.at[pl.ds(o, CHUNK)]],
                                 wv_v.at[pl.ds(o, CHUNK)], sem))
        for h in handles:
            h.wait()
        return carry

    lax.fori_loop(0, NCHUNK // 4, body, 0)
    pltpu.sync_copy(sv_v, sv_hbm.at[pl.ds(base, EPW)])
    pltpu.sync_copy(wv_v, wv_hbm.at[pl.ds(base, EPW)])


def _final_kernel(sv_ref, wv_ref, cidx_ref, xt_ref, wmax_ref,
                  dmax_ref, imax_ref):
    v = sv_ref[...]
    wv = wv_ref[...]
    ci = cidx_ref[...]
    lane = jax.lax.broadcasted_iota(jnp.int32, (Q, CAND), 1)
    slot = jax.lax.broadcasted_iota(jnp.int32, (Q, SLOT), 1)
    vacc = jnp.full((Q, SLOT), BIG, jnp.float32)
    iacc = jnp.zeros((Q, SLOT), jnp.int32)
    wacc = jnp.zeros((Q, SLOT), jnp.float32)
    for p in range(TOPK):
        m = jnp.min(v, axis=1, keepdims=True)
        am = jnp.min(jnp.where(v == m, lane, CAND), axis=1, keepdims=True)
        sel = lane == am
        iv = jnp.sum(jnp.where(sel, ci, 0), axis=1, keepdims=True)
        wvv = jnp.sum(jnp.where(sel, wv, 0.0), axis=1, keepdims=True)
        vacc = jnp.where(slot == p, m, vacc)
        iacc = jnp.where(slot == p, iv, iacc)
        wacc = jnp.where(slot == p, wvv, wacc)
        v = jnp.where(sel, BIG, v)

    xt = xt_ref[...]
    qn = jnp.sum(xt * xt, axis=1, keepdims=True)
    wmax = wmax_ref[0, 0]
    d2 = vacc + qn - (wmax - wacc) / K_CONST
    f = wacc - K_CONST * jnp.sqrt(jax.nn.relu(d2))
    f = jnp.where(slot >= TOPK, -BIG, f)
    fm = jnp.max(f, axis=1, keepdims=True)
    fam = jnp.min(jnp.where(f == fm, slot, SLOT), axis=1, keepdims=True)
    dmax_ref[...] = fm
    imax_ref[...] = jnp.sum(jnp.where(slot == fam, iacc, 0), axis=1,
                            keepdims=True)


@jax.jit
def kernel(X_tilde, X, w):
    X_tilde = X_tilde.reshape(Q, D_FEAT)
    wflat = jnp.pad(w[:, 0], (0, NP - N_DB))
    wrows = wflat.reshape(NB, 1, BN)
    wgrid = wflat.reshape(NP // D_FEAT, D_FEAT)

    wmax = pl.pallas_call(
        _wmax_kernel,
        out_shape=jax.ShapeDtypeStruct((1, 1), jnp.float32),
    )(wgrid)

    s_full, gm = pl.pallas_call(
        _score_kernel,
        grid=(NB,),
        in_specs=[
            pl.BlockSpec((Q, D_FEAT), lambda j: (0, 0)),
            pl.BlockSpec((BN, D_FEAT), lambda j: (j, 0)),
            pl.BlockSpec((1, 1, BN), lambda j: (j, 0, 0)),
            pl.BlockSpec((1, 1), lambda j: (0, 0)),
        ],
        out_specs=[
            pl.BlockSpec((Q, BN // 128, 128), lambda j: (0, j, 0)),
            pl.BlockSpec((Q, NGB), lambda j: (0, j)),
        ],
        out_shape=[
            jax.ShapeDtypeStruct((Q, NP // 128, 128), jnp.float32),
            jax.ShapeDtypeStruct((Q, NG), jnp.float32),
        ],
    )(X_tilde, X, wrows, wmax)

    sidx, cidx = pl.pallas_call(
        _group_select_kernel,
        grid=(Q // QB,),
        in_specs=[pl.BlockSpec((QB, NG), lambda i: (i, 0))],
        out_specs=[
            pl.BlockSpec((QB, CAND), lambda i: (i, 0)),
            pl.BlockSpec((QB, CAND), lambda i: (i, 0)),
        ],
        out_shape=[
            jax.ShapeDtypeStruct((Q, CAND), jnp.int32),
            jax.ShapeDtypeStruct((Q, CAND), jnp.int32),
        ],
    )(gm)

    gather = pl.kernel(
        _sc_gather,
        mesh=plsc.VectorSubcoreMesh(core_axis_name="c", subcore_axis_name="s"),
        out_type=[
            jax.ShapeDtypeStruct((Q * CAND,), jnp.float32),
            jax.ShapeDtypeStruct((Q * CAND,), jnp.float32),
        ],
        scratch_types=[
            pltpu.VMEM((EPW,), jnp.int32),
            pltpu.VMEM((EPW,), jnp.int32),
            pltpu.VMEM((EPW,), jnp.float32),
            pltpu.VMEM((EPW,), jnp.float32),
            pltpu.SemaphoreType.DMA,
        ],
    )
    sv, wv = gather(sidx.reshape(Q * CAND), cidx.reshape(Q * CAND),
                    s_full.reshape(Q * NP), wflat)

    dmax, imax = pl.pallas_call(
        _final_kernel,
        out_shape=[
            jax.ShapeDtypeStruct((Q, 1), jnp.float32),
            jax.ShapeDtypeStruct((Q, 1), jnp.int32),
        ],
    )(sv.reshape(Q, CAND), wv.reshape(Q, CAND), cidx, X_tilde, wmax)

    return dmax[:, 0], imax[:, 0]
